# baseline (speedup 1.0000x reference)
import dataclasses
import functools

import jax
import jax.numpy as jnp
from jax import lax
from jax.experimental import pallas as pl
from jax.experimental.pallas import tpu as pltpu
from jax.experimental.pallas import tpu_sc as plsc

_NUM_CORES = 2
_NUM_SUBCORES = 16
_NW = _NUM_CORES * _NUM_SUBCORES
_LANES = 16
_BLOCK = 128


@functools.lru_cache(maxsize=None)
def _build_sc_lerp_gather(t_total: int, nc: int, d: int):
    assert t_total % (_NW * _BLOCK) == 0
    rows_per_w = t_total // _NW
    nblk = rows_per_w // _BLOCK
    mesh = plsc.VectorSubcoreMesh(core_axis_name="c", subcore_axis_name="s")
    cp = pltpu.CompilerParams()
    if "needs_layout_passes" in pltpu.CompilerParams.__dataclass_fields__:
        cp = dataclasses.replace(cp, needs_layout_passes=False)

    @functools.partial(
        pl.kernel,
        out_type=jax.ShapeDtypeStruct((t_total, d), jnp.float32),
        mesh=mesh,
        compiler_params=cp,
        scratch_types=[
            pltpu.VMEM((_BLOCK,), jnp.float32),
            pltpu.VMEM((_BLOCK,), jnp.int32),
            pltpu.VMEM((_BLOCK,), jnp.int32),
            pltpu.VMEM((_BLOCK,), jnp.float32),
            pltpu.VMEM((_BLOCK, d), jnp.float32),
            pltpu.VMEM((_BLOCK, d), jnp.float32),
            pltpu.VMEM((_BLOCK, d), jnp.float32),
            pltpu.SemaphoreType.DMA,
            pltpu.SemaphoreType.DMA,
        ],
    )
    def sc_kernel(table_hbm, ts_hbm, out_hbm,
                  ts_v, idx_l, idx_r, frac_v, left_v, right_v, out_v,
                  sem_l, sem_r):
        wid = lax.axis_index("s") * _NUM_CORES + lax.axis_index("c")

        @pl.loop(0, nblk)
        def _(b):
            base = wid * rows_per_w + b * _BLOCK
            pltpu.sync_copy(ts_hbm.at[pl.ds(base, _BLOCK)], ts_v)

            @pl.loop(0, _BLOCK, step=_LANES)
            def _(i):
                tv = ts_v[pl.ds(i, _LANES)]
                idx = (tv * float(nc - 2)).astype(jnp.int32)
                idx_l[pl.ds(i, _LANES)] = idx
                idx_r[pl.ds(i, _LANES)] = idx + 1
                frac_v[pl.ds(i, _LANES)] = (
                    tv * float(nc - 1) - idx.astype(jnp.float32))

            cl = pltpu.async_copy(table_hbm.at[idx_l], left_v, sem_l)
            cr = pltpu.async_copy(table_hbm.at[idx_r], right_v, sem_r)
            cl.wait()
            cr.wait()

            @pl.loop(0, _BLOCK)
            def _(r):
                fv = plsc.load_gather(
                    frac_v, [jnp.full((_LANES,), r, jnp.int32)])
                omf = 1.0 - fv
                for c in range(0, d, _LANES):
                    lo = left_v[r, pl.ds(c, _LANES)]
                    hi = right_v[r, pl.ds(c, _LANES)]
                    out_v[r, pl.ds(c, _LANES)] = omf * lo + fv * hi

            pltpu.sync_copy(out_v, out_hbm.at[pl.ds(base, _BLOCK)])

    return sc_kernel


def kernel(timestamps, control_points):
    n_start, nc, two = control_points.shape
    t_total = timestamps.shape[0]
    d = n_start * two
    table = control_points.transpose(1, 0, 2).reshape(nc, d)
    sc_kernel = _build_sc_lerp_gather(t_total, nc, d)
    out_rows = sc_kernel(table, timestamps)
    return out_rows.reshape(t_total, n_start, two).transpose(1, 0, 2)
